# baseline (speedup 1.0000x reference)
import jax
import jax.numpy as jnp
from jax.experimental import pallas as pl
from jax.experimental.pallas import tpu as pltpu

_B, _NS, _NO = 4, 10475, 2048
_THRESHOLD = 0.1
_BIG = 1e30
_BS = 512
_NSP = ((_NS + _BS - 1) // _BS) * _BS
_NCHUNK = _NSP // _BS


def _cl_kernel(s_ref, o_ref, loss_ref, valid_ref):
    ox = o_ref[0, 0:1, :]
    oy = o_ref[0, 1:2, :]
    oz = o_ref[0, 2:3, :]
    om = o_ref[0, 3:4, :]
    o2 = ox * ox + oy * oy + oz * oz
    oxb = ox.astype(jnp.bfloat16).astype(jnp.float32)
    oyb = oy.astype(jnp.bfloat16).astype(jnp.float32)
    ozb = oz.astype(jnp.bfloat16).astype(jnp.float32)
    om2 = om * om
    omask = om > _THRESHOLD
    pcol = jnp.where(omask, 0.0, _BIG)
    no = jnp.sum(omask.astype(jnp.float32))

    def body(i, carry):
        hsum, nsum, oacc = carry
        sc = s_ref[0, pl.ds(i * _BS, _BS), :]
        sx = sc[:, 0:1]
        sy = sc[:, 1:2]
        sz = sc[:, 2:3]
        sm = sc[:, 3:4]
        s2 = sx * sx + sy * sy + sz * sz
        sm2 = sm * sm
        smask = sm > _THRESHOLD
        qcol = jnp.where(smask, 0.0, _BIG)
        sxb = sx.astype(jnp.bfloat16).astype(jnp.float32)
        syb = sy.astype(jnp.bfloat16).astype(jnp.float32)
        szb = sz.astype(jnp.bfloat16).astype(jnp.float32)
        dot = sxb * oxb + syb * oyb + szb * ozb
        t = (s2 + o2) - 2.0 * dot
        d2 = jnp.maximum(t, 1e-12)
        v1 = d2 * om2 + pcol
        rmin = jnp.min(v1, axis=1, keepdims=True)
        hsum = hsum + jnp.sum(jnp.where(smask, sm * jnp.sqrt(rmin), 0.0))
        nsum = nsum + jnp.sum(smask.astype(jnp.float32))
        v2 = d2 * sm2 + qcol
        oacc = jnp.minimum(oacc, jnp.min(v2, axis=0, keepdims=True))
        return hsum, nsum, oacc

    init = (jnp.float32(0.0), jnp.float32(0.0),
            jnp.full((1, _NO), _BIG, jnp.float32))
    hsum, ns, oacc = jax.lax.fori_loop(0, _NCHUNK, body, init)

    osum = jnp.sum(jnp.where(omask, om * jnp.sqrt(oacc), 0.0))
    h2o_mean = hsum / jnp.maximum(ns, 1.0)
    o2h_mean = osum / jnp.maximum(no, 1.0)
    valid = jnp.logical_and(ns > 0, no > 0)
    contrib = jnp.where(valid, h2o_mean + o2h_mean, 0.0)

    loss_ref[...] = contrib.reshape(1, 1, 1)
    valid_ref[...] = valid.astype(jnp.float32).reshape(1, 1, 1)


def kernel(smplx_v, object_v, smpl_occlusion_maps, object_occlusion_maps,
           smpl_mean_occlusion_map, object_mean_occlusion_map):
    sm = smpl_occlusion_maps * smpl_mean_occlusion_map[None, :]
    om = object_occlusion_maps * object_mean_occlusion_map[None, :]

    s_all = jnp.concatenate([smplx_v, sm[:, :, None]], axis=2)
    s_all = jnp.pad(s_all, ((0, 0), (0, _NSP - _NS), (0, 0)))

    o_all = jnp.concatenate(
        [object_v.transpose(0, 2, 1), om[:, None, :]], axis=1)

    loss, valid = pl.pallas_call(
        _cl_kernel,
        grid=(_B,),
        in_specs=[
            pl.BlockSpec((1, _NSP, 4), lambda b: (b, 0, 0)),
            pl.BlockSpec((1, 4, _NO), lambda b: (b, 0, 0)),
        ],
        out_specs=[
            pl.BlockSpec((1, 1, 1), lambda b: (b, 0, 0)),
            pl.BlockSpec((1, 1, 1), lambda b: (b, 0, 0)),
        ],
        out_shape=[
            jax.ShapeDtypeStruct((_B, 1, 1), jnp.float32),
            jax.ShapeDtypeStruct((_B, 1, 1), jnp.float32),
        ],
        compiler_params=pltpu.CompilerParams(
            dimension_semantics=("parallel",)),
    )(s_all, o_all)

    total = jnp.sum(loss)
    count = jnp.sum(valid)
    return jnp.where(count > 0, total / jnp.maximum(count, 1.0), total)


# speedup vs baseline: 1.1607x; 1.1607x over previous
import jax
import jax.numpy as jnp
from jax.experimental import pallas as pl
from jax.experimental.pallas import tpu as pltpu

_B, _NS, _NO = 4, 10475, 2048
_THRESHOLD = 0.1
_BIG = 1e30
_BS = 512
_NSP = ((_NS + _BS - 1) // _BS) * _BS
_NCHUNK = _NSP // _BS


def _cl_kernel(s_ref, o_ref, loss_ref, valid_ref):
    ox = o_ref[0, 0:1, :]
    oy = o_ref[0, 1:2, :]
    oz = o_ref[0, 2:3, :]
    om = o_ref[0, 3:4, :]
    o2 = ox * ox + oy * oy + oz * oz
    oxm2 = -2.0 * ox.astype(jnp.bfloat16).astype(jnp.float32)
    oym2 = -2.0 * oy.astype(jnp.bfloat16).astype(jnp.float32)
    ozm2 = -2.0 * oz.astype(jnp.bfloat16).astype(jnp.float32)
    omask = om > _THRESHOLD
    om2m = jnp.where(omask, om * om, _BIG)
    no = jnp.sum(omask.astype(jnp.float32))

    def body(i, carry):
        hsum, nsum, oacc = carry
        sc = s_ref[0, pl.ds(i * _BS, _BS), :]
        sx = sc[:, 0:1]
        sy = sc[:, 1:2]
        sz = sc[:, 2:3]
        sm = sc[:, 3:4]
        s2 = sx * sx + sy * sy + sz * sz
        smask = sm > _THRESHOLD
        sm2m = jnp.where(smask, sm * sm, _BIG)
        smw = jnp.where(smask, sm, 0.0)
        sxb = sx.astype(jnp.bfloat16).astype(jnp.float32)
        syb = sy.astype(jnp.bfloat16).astype(jnp.float32)
        szb = sz.astype(jnp.bfloat16).astype(jnp.float32)
        t = (((s2 + o2) + sxb * oxm2) + syb * oym2) + szb * ozm2
        d2 = jnp.maximum(t, 1e-12)
        v1 = d2 * om2m
        rmin = jnp.min(v1, axis=1, keepdims=True)
        hsum = hsum + jnp.sum(smw * jnp.sqrt(rmin))
        nsum = nsum + jnp.sum(smask.astype(jnp.float32))
        v2 = d2 * sm2m
        oacc = jnp.minimum(oacc, jnp.min(v2, axis=0, keepdims=True))
        return hsum, nsum, oacc

    init = (jnp.float32(0.0), jnp.float32(0.0),
            jnp.full((1, _NO), _BIG, jnp.float32))
    hsum, ns, oacc = jax.lax.fori_loop(0, _NCHUNK, body, init)

    osum = jnp.sum(jnp.where(omask, om * jnp.sqrt(oacc), 0.0))
    h2o_mean = hsum / jnp.maximum(ns, 1.0)
    o2h_mean = osum / jnp.maximum(no, 1.0)
    valid = jnp.logical_and(ns > 0, no > 0)
    contrib = jnp.where(valid, h2o_mean + o2h_mean, 0.0)

    loss_ref[...] = contrib.reshape(1, 1, 1)
    valid_ref[...] = valid.astype(jnp.float32).reshape(1, 1, 1)


def kernel(smplx_v, object_v, smpl_occlusion_maps, object_occlusion_maps,
           smpl_mean_occlusion_map, object_mean_occlusion_map):
    sm = smpl_occlusion_maps * smpl_mean_occlusion_map[None, :]
    om = object_occlusion_maps * object_mean_occlusion_map[None, :]

    s_all = jnp.concatenate([smplx_v, sm[:, :, None]], axis=2)
    s_all = jnp.pad(s_all, ((0, 0), (0, _NSP - _NS), (0, 0)))

    o_all = jnp.concatenate(
        [object_v.transpose(0, 2, 1), om[:, None, :]], axis=1)

    loss, valid = pl.pallas_call(
        _cl_kernel,
        grid=(_B,),
        in_specs=[
            pl.BlockSpec((1, _NSP, 4), lambda b: (b, 0, 0)),
            pl.BlockSpec((1, 4, _NO), lambda b: (b, 0, 0)),
        ],
        out_specs=[
            pl.BlockSpec((1, 1, 1), lambda b: (b, 0, 0)),
            pl.BlockSpec((1, 1, 1), lambda b: (b, 0, 0)),
        ],
        out_shape=[
            jax.ShapeDtypeStruct((_B, 1, 1), jnp.float32),
            jax.ShapeDtypeStruct((_B, 1, 1), jnp.float32),
        ],
        compiler_params=pltpu.CompilerParams(
            dimension_semantics=("parallel",)),
    )(s_all, o_all)

    total = jnp.sum(loss)
    count = jnp.sum(valid)
    return jnp.where(count > 0, total / jnp.maximum(count, 1.0), total)


# speedup vs baseline: 1.1850x; 1.0210x over previous
import jax
import jax.numpy as jnp
from jax.experimental import pallas as pl
from jax.experimental.pallas import tpu as pltpu

_B, _NS, _NO = 4, 10475, 2048
_THRESHOLD = 0.1
_BIG = 1e30
_BS = 512
_NSP = ((_NS + _BS - 1) // _BS) * _BS
_NCHUNK = _NSP // _BS


def _cl_kernel(s_ref, o_ref, loss_ref, valid_ref):
    ox = o_ref[0, 0:1, :]
    oy = o_ref[0, 1:2, :]
    oz = o_ref[0, 2:3, :]
    om = o_ref[0, 3:4, :]
    o2 = ox * ox + oy * oy + oz * oz
    oxm2 = -2.0 * ox.astype(jnp.bfloat16).astype(jnp.float32)
    oym2 = -2.0 * oy.astype(jnp.bfloat16).astype(jnp.float32)
    ozm2 = -2.0 * oz.astype(jnp.bfloat16).astype(jnp.float32)
    omask = om > _THRESHOLD
    om2m = jnp.where(omask, om * om, _BIG)
    no = jnp.sum(omask.astype(jnp.float32))

    def body(i, carry):
        hsum, nsum, oacc = carry
        sc = s_ref[0, pl.ds(i * _BS, _BS), :]
        sx = sc[:, 0:1]
        sy = sc[:, 1:2]
        sz = sc[:, 2:3]
        sm = sc[:, 3:4]
        s2 = sx * sx + sy * sy + sz * sz
        smask = sm > _THRESHOLD
        sm2m = jnp.where(smask, sm * sm, _BIG)
        smw = jnp.where(smask, sm, 0.0)
        sxb = sx.astype(jnp.bfloat16).astype(jnp.float32)
        syb = sy.astype(jnp.bfloat16).astype(jnp.float32)
        szb = sz.astype(jnp.bfloat16).astype(jnp.float32)
        t = (((s2 + o2) + sxb * oxm2) + syb * oym2) + szb * ozm2
        d2 = jnp.maximum(t, 1e-12)
        v1 = d2 * om2m
        rmin = jnp.min(v1, axis=1, keepdims=True)
        hsum = hsum + smw * jnp.sqrt(rmin)
        nsum = nsum + smask.astype(jnp.float32)
        v2 = d2 * sm2m
        oacc = jnp.minimum(oacc, jnp.min(v2, axis=0, keepdims=True))
        return hsum, nsum, oacc

    init = (jnp.zeros((_BS, 1), jnp.float32), jnp.zeros((_BS, 1), jnp.float32),
            jnp.full((1, _NO), _BIG, jnp.float32))
    hvec, nvec, oacc = jax.lax.fori_loop(0, _NCHUNK, body, init)
    hsum = jnp.sum(hvec)
    ns = jnp.sum(nvec)

    osum = jnp.sum(jnp.where(omask, om * jnp.sqrt(oacc), 0.0))
    h2o_mean = hsum / jnp.maximum(ns, 1.0)
    o2h_mean = osum / jnp.maximum(no, 1.0)
    valid = jnp.logical_and(ns > 0, no > 0)
    contrib = jnp.where(valid, h2o_mean + o2h_mean, 0.0)

    loss_ref[...] = contrib.reshape(1, 1, 1)
    valid_ref[...] = valid.astype(jnp.float32).reshape(1, 1, 1)


def kernel(smplx_v, object_v, smpl_occlusion_maps, object_occlusion_maps,
           smpl_mean_occlusion_map, object_mean_occlusion_map):
    sm = smpl_occlusion_maps * smpl_mean_occlusion_map[None, :]
    om = object_occlusion_maps * object_mean_occlusion_map[None, :]

    s_all = jnp.concatenate([smplx_v, sm[:, :, None]], axis=2)
    s_all = jnp.pad(s_all, ((0, 0), (0, _NSP - _NS), (0, 0)))

    o_all = jnp.concatenate(
        [object_v.transpose(0, 2, 1), om[:, None, :]], axis=1)

    loss, valid = pl.pallas_call(
        _cl_kernel,
        grid=(_B,),
        in_specs=[
            pl.BlockSpec((1, _NSP, 4), lambda b: (b, 0, 0)),
            pl.BlockSpec((1, 4, _NO), lambda b: (b, 0, 0)),
        ],
        out_specs=[
            pl.BlockSpec((1, 1, 1), lambda b: (b, 0, 0)),
            pl.BlockSpec((1, 1, 1), lambda b: (b, 0, 0)),
        ],
        out_shape=[
            jax.ShapeDtypeStruct((_B, 1, 1), jnp.float32),
            jax.ShapeDtypeStruct((_B, 1, 1), jnp.float32),
        ],
        compiler_params=pltpu.CompilerParams(
            dimension_semantics=("parallel",)),
    )(s_all, o_all)

    total = jnp.sum(loss)
    count = jnp.sum(valid)
    return jnp.where(count > 0, total / jnp.maximum(count, 1.0), total)
